# baseline (speedup 1.0000x reference)
import jax
import jax.numpy as jnp
from jax import lax
from jax.experimental import pallas as pl
from jax.experimental.pallas import tpu as pltpu
from jax.experimental.pallas import tpu_sc as plsc

NUM_CLASSES = 1000000
EMBED_DIM = 64
MODEL_DIM = 128
BATCH = 16384

_NC = 2
_NS = 16
_NW = _NC * _NS
_TPW = 248
_NWIN = _TPW // 4
_TCMAX = 7812
_TAIL0 = _TCMAX * 128
_OPAD = 2048
_LCAP = BATCH + 16 * 64


def _sc_body(labels_hbm, tableT_hbm, x3_hbm, all_lab, cpk, bpk,
             wcnt, woffA, woffB, tmp16, win0, win1, slab, sidx,
             st_ref, fsem, wsem):
    wid = lax.axis_index("s") * _NC + lax.axis_index("c")
    pltpu.sync_copy(labels_hbm.at[:], all_lab)
    iota16 = lax.iota(jnp.int32, 16)
    ones16 = jnp.ones((16,), jnp.int32)
    LO = wid * _TPW
    HI = jnp.minimum(LO + _TPW, _TCMAX)
    st_ref[0] = 0
    st_ref[1] = 0

    def _fire(w, buf):
        for a in range(8):
            pltpu.async_copy(
                tableT_hbm.at[pl.ds(8 * a, 8), pl.ds((LO + 4 * w) * 128, 512)],
                buf.at[pl.ds(8 * a, 8), :], fsem,
            )

    @pl.when(LO + 0 < HI)
    def _():
        _fire(0, win0)

    @pl.when(LO + 4 < HI)
    def _():
        _fire(1, win1)

    for j in range(4):
        wcnt[pl.ds(16 * j, 16)] = jnp.zeros((16,), jnp.int32)

    def _compress(v, off):
        labs = all_lab[pl.ds(16 * v, 16)]
        wtc = (labs >> 7) - LO
        m = (wtc >= 0) & (wtc < (HI - LO)) & (labs < _TAIL0)
        lib = (labs & 127) + ((wtc & 3) << 7)
        pk = ((wtc >> 2) << 23) | (lib << 14) | (16 * v + iota16)
        plsc.store_compressed(cpk.at[pl.ds(off, 16)], pk, mask=m)
        wv = jnp.where(m, wtc >> 2, 63)
        plsc.addupdate_scatter(wcnt, [wv], ones16, mask=m)
        return off + jnp.max(plsc.all_reduce_population_count(m))

    L = pl.loop(0, BATCH // 16, init_carry=jnp.int32(0))(_compress)
    nvec = (L + 15) >> 4

    def _prefix(j, carry):
        cnt = wcnt[pl.ds(16 * j, 16)]
        pc = (cnt + 15) & ~15
        ic = plsc.cumsum(pc)
        woffA[pl.ds(16 * j, 16)] = ic - pc + carry
        woffB[pl.ds(16 * j, 16)] = ic - pc + carry
        return carry + jnp.max(ic)

    pl.loop(0, 4, init_carry=jnp.int32(0))(_prefix)

    def _bucket(v):
        pk = cpk[pl.ds(16 * v, 16)]
        wv = pk >> 23
        valid = (16 * v + iota16) < L
        wv = jnp.where(valid, wv, 63)
        sk, sl = plsc.sort_key_val(wv, iota16)
        tmp16[pl.ds(0, 16)] = sk
        prev = plsc.load_gather(tmp16, [jnp.maximum(iota16 - 1, 0)])
        isst = (sk != prev) | (iota16 == 0)
        rank_s = iota16 - plsc.cummax(jnp.where(isst, iota16, 0))
        plsc.store_scatter(tmp16, [sl], rank_s)
        ranks = tmp16[pl.ds(0, 16)]
        base = plsc.load_gather(woffB, [wv])
        slots = base + ranks
        plsc.store_scatter(bpk, [slots], pk, mask=valid)
        plsc.addupdate_scatter(woffB, [wv], ones16, mask=valid)

    pl.loop(0, nvec)(_bucket)

    def _chunk_extract(libv, posv, m, buf):
        libv = jnp.where(m, libv, 0)

        @pl.when(st_ref[0] != 0)
        def _():
            pltpu.make_async_copy(slab, x3_hbm.at[sidx], wsem).wait()

        for c in range(EMBED_DIM):
            vals = plsc.load_gather(
                buf, [jnp.zeros((16,), jnp.int32) + c, libv]
            )
            plsc.store_scatter(
                slab, [iota16, jnp.zeros((16,), jnp.int32),
                       jnp.zeros((16,), jnp.int32) + c],
                vals, mask=m,
            )
        dummy = BATCH + ((wid * 64 + st_ref[1] + iota16) % _OPAD)
        sidx[pl.ds(0, 16)] = jnp.where(m, posv, dummy)
        pltpu.async_copy(slab, x3_hbm.at[sidx], wsem)
        st_ref[0] = 1
        st_ref[1] = st_ref[1] + 16

    def _scan(w, buf):
        ws = woffA[pl.ds(w & ~15, 16)]
        wc = wcnt[pl.ds(w & ~15, 16)]
        lsel = iota16 == (w & 15)
        s = jnp.max(jnp.where(lsel, ws, 0))
        cnt = jnp.max(jnp.where(lsel, wc, 0))

        def _chunkloop(k):
            pk = bpk[pl.ds(s + 16 * k, 16)]
            m = iota16 < (cnt - 16 * k)
            _chunk_extract((pk >> 14) & 511, pk & 16383, m, buf)

        pl.loop(0, (cnt + 15) >> 4)(_chunkloop)

    def _wait_win(buf):
        pltpu.make_async_copy(
            tableT_hbm.at[:, pl.ds(0, 512)], buf, fsem
        ).wait()

    def _window(w):
        even = (w % 2) == 0
        valid = LO + 4 * w < HI

        @pl.when(even & valid)
        def _():
            _wait_win(win0)
            _scan(w, win0)

        @pl.when((~even) & valid)
        def _():
            _wait_win(win1)
            _scan(w, win1)

        nxt = LO + 4 * (w + 2) < HI

        @pl.when(even & nxt)
        def _():
            _fire(w + 2, win0)

        @pl.when((~even) & nxt)
        def _():
            _fire(w + 2, win1)

    pl.loop(0, _NWIN)(_window)

    @pl.when(wid == _NW - 1)
    def _():
        tailbuf = win0
        tcopies = []
        for r in range(EMBED_DIM):
            tcopies.append(
                pltpu.async_copy(
                    tableT_hbm.at[r].at[pl.ds(_TAIL0, 64)],
                    tailbuf.at[r, pl.ds(0, 64)], fsem,
                )
            )
        for c in tcopies:
            c.wait()

        def _tvec(v):
            labs = all_lab[pl.ds(16 * v, 16)]
            m = labs >= _TAIL0

            @pl.when(jnp.any(m))
            def _():
                _chunk_extract(labs - _TAIL0, 16 * v + iota16, m, tailbuf)

        pl.loop(0, BATCH // 16)(_tvec)

    @pl.when(st_ref[0] != 0)
    def _():
        pltpu.make_async_copy(slab, x3_hbm.at[sidx], wsem).wait()


@jax.jit
def _sc_gather(labels, tableT):
    mesh = plsc.VectorSubcoreMesh(core_axis_name="c", subcore_axis_name="s")
    return pl.kernel(
        _sc_body,
        out_type=jax.ShapeDtypeStruct((BATCH + _OPAD, 1, 128), jnp.float32),
        mesh=mesh,
        scratch_types=[
            pltpu.VMEM((BATCH,), jnp.int32),
            pltpu.VMEM((BATCH + 16,), jnp.int32),
            pltpu.VMEM((_LCAP,), jnp.int32),
            pltpu.VMEM((64,), jnp.int32),
            pltpu.VMEM((64,), jnp.int32),
            pltpu.VMEM((64,), jnp.int32),
            pltpu.VMEM((16,), jnp.int32),
            pltpu.VMEM((EMBED_DIM, 512), jnp.float32),
            pltpu.VMEM((EMBED_DIM, 512), jnp.float32),
            pltpu.VMEM((16, 1, 128), jnp.float32),
            pltpu.VMEM((16,), jnp.int32),
            pltpu.SMEM((2,), jnp.int32),
            pltpu.SemaphoreType.DMA,
            pltpu.SemaphoreType.DMA,
        ],
        compiler_params=pltpu.CompilerParams(needs_layout_passes=False),
    )(labels, tableT)


def _mlp_body(x_ref, w1_ref, b1_ref, w2_ref, b2_ref, o_ref):
    x = x_ref[...][:, :EMBED_DIM]
    h = jnp.dot(x, w1_ref[...], preferred_element_type=jnp.float32)
    h = h + b1_ref[...]
    h = h * jax.nn.sigmoid(h)
    o = jnp.dot(h, w2_ref[...], preferred_element_type=jnp.float32)
    o_ref[...] = o + b2_ref[...]


_MLP_BLOCK = 2048


@jax.jit
def _tc_mlp(x2, W1, b1, W2, b2):
    grid = (BATCH // _MLP_BLOCK,)
    return pl.pallas_call(
        _mlp_body,
        grid=grid,
        in_specs=[
            pl.BlockSpec((_MLP_BLOCK, 128), lambda i: (i, 0)),
            pl.BlockSpec((EMBED_DIM, MODEL_DIM), lambda i: (0, 0)),
            pl.BlockSpec((1, MODEL_DIM), lambda i: (0, 0)),
            pl.BlockSpec((MODEL_DIM, MODEL_DIM), lambda i: (0, 0)),
            pl.BlockSpec((1, MODEL_DIM), lambda i: (0, 0)),
        ],
        out_specs=pl.BlockSpec((_MLP_BLOCK, MODEL_DIM), lambda i: (i, 0)),
        out_shape=jax.ShapeDtypeStruct((BATCH, MODEL_DIM), jnp.float32),
    )(x2, W1, b1, W2, b2)


def kernel(labels, table, W1, b1, W2, b2):
    x3 = _sc_gather(labels.astype(jnp.int32), table.T)
    x2 = x3.reshape(BATCH + _OPAD, 128)
    return _tc_mlp(x2, W1, b1.reshape(1, MODEL_DIM), W2,
                   b2.reshape(1, MODEL_DIM))


# speedup vs baseline: 1.0040x; 1.0040x over previous
import jax
import jax.numpy as jnp
from jax import lax
from jax.experimental import pallas as pl
from jax.experimental.pallas import tpu as pltpu
from jax.experimental.pallas import tpu_sc as plsc

NUM_CLASSES = 1000000
EMBED_DIM = 64
MODEL_DIM = 128
BATCH = 16384

_NC = 2
_NS = 16
_NW = _NC * _NS
_TPW = 248
_NWIN = _TPW // 4
_TCMAX = 7812
_TAIL0 = _TCMAX * 128
_OPAD = 2048
_LCAP = BATCH + 16 * 64


def _sc_body(labels_hbm, tableT_hbm, x3_hbm, all_lab, cpk, bpk,
             wcnt, woffA, woffB, tmp16, win0, win1, slab, sidx,
             st_ref, fsem, wsem):
    wid = lax.axis_index("s") * _NC + lax.axis_index("c")
    pltpu.sync_copy(labels_hbm.at[:], all_lab)
    iota16 = lax.iota(jnp.int32, 16)
    ones16 = jnp.ones((16,), jnp.int32)
    LO = wid * _TPW
    HI = jnp.minimum(LO + _TPW, _TCMAX)
    st_ref[0] = 0
    st_ref[1] = 0

    def _fire(w, buf):
        for a in range(8):
            pltpu.async_copy(
                tableT_hbm.at[pl.ds(8 * a, 8),
                              pl.ds((LO + 4 * w) * 128, 512)],
                buf.at[pl.ds(8 * a, 8), :], fsem,
            )

    @pl.when(LO + 0 < HI)
    def _():
        _fire(0, win0)

    @pl.when(LO + 4 < HI)
    def _():
        _fire(1, win1)

    for j in range(4):
        wcnt[pl.ds(16 * j, 16)] = jnp.zeros((16,), jnp.int32)

    def _one(base, labs):
        wtc = (labs >> 7) - LO
        m = (wtc >= 0) & (wtc < (HI - LO)) & (labs < _TAIL0)
        lib = (labs & 127) + ((wtc & 3) << 7)
        pk = ((wtc >> 2) << 23) | (lib << 14) | (base + iota16)
        wv = jnp.where(m, wtc >> 2, 63)
        return m, pk, wv

    def _compress(v, off):
        labs0 = all_lab[pl.ds(32 * v, 16)]
        labs1 = all_lab[pl.ds(32 * v + 16, 16)]
        m0, pk0, wv0 = _one(32 * v, labs0)
        m1, pk1, wv1 = _one(32 * v + 16, labs1)
        n0 = plsc.all_reduce_population_count(m0)
        n1 = plsc.all_reduce_population_count(m1)
        plsc.store_compressed(cpk.at[pl.ds(off, 16)], pk0, mask=m0)
        plsc.addupdate_scatter(wcnt, [wv0], ones16, mask=m0)
        off1 = off + jnp.max(n0)
        plsc.store_compressed(cpk.at[pl.ds(off1, 16)], pk1, mask=m1)
        plsc.addupdate_scatter(wcnt, [wv1], ones16, mask=m1)
        return off1 + jnp.max(n1)

    L = pl.loop(0, BATCH // 32, init_carry=jnp.int32(0))(_compress)
    nvec = (L + 15) >> 4

    def _prefix(j, carry):
        cnt = wcnt[pl.ds(16 * j, 16)]
        pc = (cnt + 15) & ~15
        ic = plsc.cumsum(pc)
        woffA[pl.ds(16 * j, 16)] = ic - pc + carry
        woffB[pl.ds(16 * j, 16)] = ic - pc + carry
        return carry + jnp.max(ic)

    pl.loop(0, 4, init_carry=jnp.int32(0))(_prefix)

    def _bucket(v):
        pk = cpk[pl.ds(16 * v, 16)]
        wv = pk >> 23
        valid = (16 * v + iota16) < L
        wv = jnp.where(valid, wv, 63)
        sk, sl = plsc.sort_key_val(wv, iota16)
        tmp16[pl.ds(0, 16)] = sk
        prev = plsc.load_gather(tmp16, [jnp.maximum(iota16 - 1, 0)])
        isst = (sk != prev) | (iota16 == 0)
        rank_s = iota16 - plsc.cummax(jnp.where(isst, iota16, 0))
        plsc.store_scatter(tmp16, [sl], rank_s)
        ranks = tmp16[pl.ds(0, 16)]
        base = plsc.load_gather(woffB, [wv])
        slots = base + ranks
        plsc.store_scatter(bpk, [slots], pk, mask=valid)
        plsc.addupdate_scatter(woffB, [wv], ones16, mask=valid)

    pl.loop(0, nvec)(_bucket)

    def _chunk_extract(libv, posv, m, buf):
        libv = jnp.where(m, libv, 0)

        @pl.when(st_ref[0] != 0)
        def _():
            pltpu.make_async_copy(slab, x3_hbm.at[sidx], wsem).wait()

        for c in range(EMBED_DIM):
            vals = plsc.load_gather(
                buf, [jnp.zeros((16,), jnp.int32) + c, libv]
            )
            plsc.store_scatter(
                slab, [iota16, jnp.zeros((16,), jnp.int32),
                       jnp.zeros((16,), jnp.int32) + c],
                vals, mask=m,
            )
        dummy = BATCH + ((wid * 64 + st_ref[1] + iota16) % _OPAD)
        sidx[pl.ds(0, 16)] = jnp.where(m, posv, dummy)
        pltpu.async_copy(slab, x3_hbm.at[sidx], wsem)
        st_ref[0] = 1
        st_ref[1] = st_ref[1] + 16

    def _scan(w, buf):
        ws = woffA[pl.ds(w & ~15, 16)]
        wc = wcnt[pl.ds(w & ~15, 16)]
        lsel = iota16 == (w & 15)
        s = jnp.max(jnp.where(lsel, ws, 0))
        cnt = jnp.max(jnp.where(lsel, wc, 0))

        def _chunkloop(k):
            pk = bpk[pl.ds(s + 16 * k, 16)]
            m = iota16 < (cnt - 16 * k)
            _chunk_extract((pk >> 14) & 511, pk & 16383, m, buf)

        pl.loop(0, (cnt + 15) >> 4)(_chunkloop)

    def _wait_win(buf):
        pltpu.make_async_copy(
            tableT_hbm.at[:, pl.ds(0, 512)], buf, fsem
        ).wait()

    def _window(w):
        even = (w % 2) == 0
        valid = LO + 4 * w < HI

        @pl.when(even & valid)
        def _():
            _wait_win(win0)
            _scan(w, win0)

        @pl.when((~even) & valid)
        def _():
            _wait_win(win1)
            _scan(w, win1)

        nxt = LO + 4 * (w + 2) < HI

        @pl.when(even & nxt)
        def _():
            _fire(w + 2, win0)

        @pl.when((~even) & nxt)
        def _():
            _fire(w + 2, win1)

    pl.loop(0, _NWIN)(_window)

    @pl.when(wid == _NW - 1)
    def _():
        tailbuf = win0
        tcopies = []
        for r in range(EMBED_DIM):
            tcopies.append(
                pltpu.async_copy(
                    tableT_hbm.at[r].at[pl.ds(_TAIL0, 64)],
                    tailbuf.at[r, pl.ds(0, 64)], fsem,
                )
            )
        for c in tcopies:
            c.wait()

        def _tvec(v):
            labs = all_lab[pl.ds(16 * v, 16)]
            m = labs >= _TAIL0

            @pl.when(jnp.any(m))
            def _():
                _chunk_extract(labs - _TAIL0, 16 * v + iota16, m, tailbuf)

        pl.loop(0, BATCH // 16)(_tvec)

    @pl.when(st_ref[0] != 0)
    def _():
        pltpu.make_async_copy(slab, x3_hbm.at[sidx], wsem).wait()


@jax.jit
def _sc_gather(labels, tableT):
    mesh = plsc.VectorSubcoreMesh(core_axis_name="c", subcore_axis_name="s")
    return pl.kernel(
        _sc_body,
        out_type=jax.ShapeDtypeStruct((BATCH + _OPAD, 1, 128), jnp.float32),
        mesh=mesh,
        scratch_types=[
            pltpu.VMEM((BATCH,), jnp.int32),
            pltpu.VMEM((BATCH + 16,), jnp.int32),
            pltpu.VMEM((_LCAP,), jnp.int32),
            pltpu.VMEM((64,), jnp.int32),
            pltpu.VMEM((64,), jnp.int32),
            pltpu.VMEM((64,), jnp.int32),
            pltpu.VMEM((16,), jnp.int32),
            pltpu.VMEM((EMBED_DIM, 512), jnp.float32),
            pltpu.VMEM((EMBED_DIM, 512), jnp.float32),
            pltpu.VMEM((16, 1, 128), jnp.float32),
            pltpu.VMEM((16,), jnp.int32),
            pltpu.SMEM((2,), jnp.int32),
            pltpu.SemaphoreType.DMA,
            pltpu.SemaphoreType.DMA,
        ],
        compiler_params=pltpu.CompilerParams(needs_layout_passes=False),
    )(labels, tableT)


def _mlp_body(x_ref, w1_ref, b1_ref, w2_ref, b2_ref, o_ref):
    x = x_ref[...][:, :EMBED_DIM]
    h = jnp.dot(x, w1_ref[...], preferred_element_type=jnp.float32)
    h = h + b1_ref[...]
    h = h * jax.nn.sigmoid(h)
    o = jnp.dot(h, w2_ref[...], preferred_element_type=jnp.float32)
    o_ref[...] = o + b2_ref[...]


_MLP_BLOCK = 2048


@jax.jit
def _tc_mlp(x2, W1, b1, W2, b2):
    grid = (BATCH // _MLP_BLOCK,)
    return pl.pallas_call(
        _mlp_body,
        grid=grid,
        in_specs=[
            pl.BlockSpec((_MLP_BLOCK, 128), lambda i: (i, 0)),
            pl.BlockSpec((EMBED_DIM, MODEL_DIM), lambda i: (0, 0)),
            pl.BlockSpec((1, MODEL_DIM), lambda i: (0, 0)),
            pl.BlockSpec((MODEL_DIM, MODEL_DIM), lambda i: (0, 0)),
            pl.BlockSpec((1, MODEL_DIM), lambda i: (0, 0)),
        ],
        out_specs=pl.BlockSpec((_MLP_BLOCK, MODEL_DIM), lambda i: (i, 0)),
        out_shape=jax.ShapeDtypeStruct((BATCH, MODEL_DIM), jnp.float32),
    )(x2, W1, b1, W2, b2)


def kernel(labels, table, W1, b1, W2, b2):
    x3 = _sc_gather(labels.astype(jnp.int32), table.T)
    x2 = x3.reshape(BATCH + _OPAD, 128)
    return _tc_mlp(x2, W1, b1.reshape(1, MODEL_DIM), W2,
                   b2.reshape(1, MODEL_DIM))
